# baseline (speedup 1.0000x reference)
import functools

import jax
import jax.numpy as jnp
from jax import lax
from jax.experimental import pallas as pl
from jax.experimental.pallas import tpu as pltpu
from jax.experimental.pallas import tpu_sc as plsc

N_RAYS = 131072
NWEIGHT = 127
NBIN = 128
NSAMP = 128
NV = NBIN // 16

NUM_CORES = 2
NUM_SUBCORES = 16
NUM_W = NUM_CORES * NUM_SUBCORES
RAYS_PER_W = N_RAYS // NUM_W
CHUNK = 64
NCHUNK = RAYS_PER_W // CHUNK

HSTRIDE = 144

_GDN = lax.GatherDimensionNumbers(
    offset_dims=(), collapsed_slice_dims=(0,), start_index_map=(0,))


def _perm(x, idx):
    return lax.gather(x, idx[:, None], _GDN, (1,),
                      mode=lax.GatherScatterMode.PROMISE_IN_BOUNDS)


def _ray_prep(r, wv, bv, sv, av, hv):
    lane = lax.iota(jnp.int32, 16)
    zero_v = jnp.zeros((16,), jnp.float32)
    lane15 = jnp.full((16,), 15, jnp.int32)
    lane0 = jnp.zeros((16,), jnp.int32)
    lanem1 = jnp.maximum(lane - 1, 0)
    lanep1 = jnp.minimum(lane + 1, 15)

    wbase = r * NWEIGHT
    rrow = r * NBIN
    hbase = r * HSTRIDE

    w = [wv[pl.ds(wbase + 16 * b, 16)] + 1e-5 for b in range(NV)]
    w[NV - 1] = jnp.where(lane < 15, w[NV - 1], 0.0)

    cum = [plsc.cumsum(w[b]) for b in range(NV)]
    s = [_perm(cum[b], lane15) for b in range(NV)]
    off = [zero_v]
    for b in range(NV - 1):
        off.append(off[b] + s[b])
    tot_v = off[NV - 1] + s[NV - 1]
    inv_v = 1.0 / tot_v

    bn = [bv[pl.ds(rrow + 16 * b, 16)] for b in range(NV)]

    zi = jnp.zeros((16,), jnp.int32)
    for q in range(9):
        hv[pl.ds(hbase + 16 * q, 16)] = zi

    hrow = hv.at[pl.ds(hbase, HSTRIDE)]
    ones = jnp.ones((16,), jnp.int32)
    for b in range(NV):
        cn = (cum[b] + off[b]) * inv_v
        offinv = off[b] * inv_v
        c0 = jnp.where(lane == 0, offinv, _perm(cn, lanem1))
        b1 = _perm(bn[b], lanep1)
        if b < NV - 1:
            b1 = jnp.where(lane == 15, _perm(bn[b + 1], lane0), b1)
        dC = w[b] * inv_v
        dB = b1 - bn[b]
        S = jnp.where(dC < 1e-5, dB, dB / dC)
        A = bn[b] - c0 * S
        sv[pl.ds(rrow + 16 * b, 16)] = S
        av[pl.ds(rrow + 16 * b, 16)] = A
        kb = (cn * 128.0 + 0.49999997).astype(jnp.int32)
        plsc.addupdate_scatter(hrow, [kb], ones)


def _ray_sample(r, ov, sv, av, hv):
    lane = lax.iota(jnp.int32, 16)
    lane_f = lane.astype(jnp.float32)
    lane15 = jnp.full((16,), 15, jnp.int32)

    rrow = r * NBIN
    hbase = r * HSTRIDE

    hcum = []
    for b in range(NV):
        hb = hv[pl.ds(hbase + 16 * b, 16)]
        hcum.append(plsc.cumsum(hb))
    hs = [_perm(hcum[b], lane15) for b in range(NV)]
    hoff = [jnp.zeros((16,), jnp.int32)]
    for b in range(NV - 1):
        hoff.append(hoff[b] + hs[b])

    srow_ref = sv.at[pl.ds(rrow, NBIN)]
    arow_ref = av.at[pl.ds(rrow, NBIN)]
    for b in range(NV):
        below = hcum[b] + hoff[b]
        Sg = plsc.load_gather(srow_ref, [below])
        Ag = plsc.load_gather(arow_ref, [below])
        ub = (lane_f + (16.0 * b + 0.5)) * (1.0 / 128.0)
        ov[pl.ds(r * NSAMP + 16 * b, 16)] = Ag + ub * Sg


@functools.partial(
    pl.kernel,
    out_type=jax.ShapeDtypeStruct((N_RAYS * NSAMP,), jnp.float32),
    mesh=plsc.VectorSubcoreMesh(core_axis_name="c", subcore_axis_name="s"),
    compiler_params=pltpu.CompilerParams(needs_layout_passes=False),
    scratch_types=[
        pltpu.VMEM((CHUNK * NWEIGHT + 16,), jnp.float32),
        pltpu.VMEM((CHUNK * NWEIGHT + 16,), jnp.float32),
        pltpu.VMEM((CHUNK * NBIN,), jnp.float32),
        pltpu.VMEM((CHUNK * NBIN,), jnp.float32),
        pltpu.VMEM((CHUNK * NSAMP,), jnp.float32),
        pltpu.VMEM((CHUNK * NSAMP,), jnp.float32),
        pltpu.VMEM((CHUNK * NBIN,), jnp.float32),
        pltpu.VMEM((CHUNK * NBIN,), jnp.float32),
        pltpu.VMEM((CHUNK * HSTRIDE,), jnp.int32),
        pltpu.SemaphoreType.DMA,
        pltpu.SemaphoreType.DMA,
        pltpu.SemaphoreType.DMA,
        pltpu.SemaphoreType.DMA,
    ],
)
def _sample_pdf_sc(b_hbm, w_hbm, o_hbm, wv0, wv1, bv0, bv1, ov0, ov1,
                   sv, av, hv, si0, si1, so0, so1):
    wid = lax.axis_index("s") * NUM_CORES + lax.axis_index("c")
    tile_base = wid * RAYS_PER_W
    bufs = ((wv0, bv0, ov0, si0, so0), (wv1, bv1, ov1, si1, so1))

    def in_slices(c):
        rbase = tile_base + c * CHUNK
        return (w_hbm.at[pl.ds(rbase * NWEIGHT, CHUNK * NWEIGHT)],
                b_hbm.at[pl.ds(rbase * NBIN, CHUNK * NBIN)])

    def out_slice(c):
        rbase = tile_base + c * CHUNK
        return o_hbm.at[pl.ds(rbase * NSAMP, CHUNK * NSAMP)]

    def start_in(c, wvx, bvx, sin):
        wh, bh = in_slices(c)
        pltpu.async_copy(wh, wvx.at[pl.ds(0, CHUNK * NWEIGHT)], sin)
        pltpu.async_copy(bh, bvx, sin)

    def wait_in(c, wvx, bvx, sin):
        wh, bh = in_slices(c)
        pltpu.make_async_copy(wh, wvx.at[pl.ds(0, CHUNK * NWEIGHT)], sin).wait()
        pltpu.make_async_copy(bh, bvx, sin).wait()

    start_in(0, wv0, bv0, si0)
    start_in(1, wv1, bv1, si1)

    def pair_body(g, carry):
        for par, (wvx, bvx, ovx, sin, sout) in enumerate(bufs):
            c = 2 * g + par
            wait_in(c, wvx, bvx, sin)

            @pl.when(c >= 2)
            def _drain_prev_out():
                pltpu.make_async_copy(ovx, out_slice(c - 2), sout).wait()

            @plsc.parallel_loop(0, CHUNK, unroll=4)
            def _rays_a(r):
                _ray_prep(r, wvx, bvx, sv, av, hv)

            @plsc.parallel_loop(0, CHUNK, unroll=8)
            def _rays_b(r):
                _ray_sample(r, ovx, sv, av, hv)

            pltpu.async_copy(ovx, out_slice(c), sout)

            @pl.when(c + 2 < NCHUNK)
            def _prefetch_next():
                start_in(c + 2, wvx, bvx, sin)
        return carry

    lax.fori_loop(0, NCHUNK // 2, pair_body, 0)

    pltpu.make_async_copy(ov0, out_slice(NCHUNK - 2), so0).wait()
    pltpu.make_async_copy(ov1, out_slice(NCHUNK - 1), so1).wait()


def kernel(bins, weights, n_samples):
    del n_samples
    out = _sample_pdf_sc(bins.reshape(-1), weights.reshape(-1))
    return out.reshape(N_RAYS, NSAMP)


# speedup vs baseline: 1.0746x; 1.0746x over previous
import functools

import jax
import jax.numpy as jnp
from jax import lax
from jax.experimental import pallas as pl
from jax.experimental.pallas import tpu as pltpu
from jax.experimental.pallas import tpu_sc as plsc

N_RAYS = 131072
NWEIGHT = 127
NBIN = 128
NSAMP = 128
NV = NBIN // 16

NUM_CORES = 2
NUM_SUBCORES = 16
NUM_W = NUM_CORES * NUM_SUBCORES
RAYS_PER_W = N_RAYS // NUM_W
CHUNK = 64
NCHUNK = RAYS_PER_W // CHUNK

HSTRIDE = 144

_GDN = lax.GatherDimensionNumbers(
    offset_dims=(), collapsed_slice_dims=(0,), start_index_map=(0,))


def _perm(x, idx):
    return lax.gather(x, idx[:, None], _GDN, (1,),
                      mode=lax.GatherScatterMode.PROMISE_IN_BOUNDS)


def _ray_prep(r, wv, bv, sv, av, hv):
    lane = lax.iota(jnp.int32, 16)
    zero_v = jnp.zeros((16,), jnp.float32)
    lane15 = jnp.full((16,), 15, jnp.int32)
    lane0 = jnp.zeros((16,), jnp.int32)
    lanem1 = jnp.maximum(lane - 1, 0)
    lanep1 = jnp.minimum(lane + 1, 15)

    wbase = r * NWEIGHT
    rrow = r * NBIN
    hbase = r * HSTRIDE

    w = [wv[pl.ds(wbase + 16 * b, 16)] + 1e-5 for b in range(NV)]
    w[NV - 1] = jnp.where(lane < 15, w[NV - 1], 0.0)

    cum = [plsc.cumsum(w[b]) for b in range(NV)]
    s = [_perm(cum[b], lane15) for b in range(NV)]
    off = [zero_v]
    for b in range(NV - 1):
        off.append(off[b] + s[b])
    tot_v = off[NV - 1] + s[NV - 1]
    inv_v = 1.0 / tot_v

    bn = [bv[pl.ds(rrow + 16 * b, 16)] for b in range(NV)]

    zi = jnp.zeros((16,), jnp.int32)
    for q in range(9):
        hv[pl.ds(hbase + 16 * q, 16)] = zi

    hrow = hv.at[pl.ds(hbase, HSTRIDE)]
    ones = jnp.ones((16,), jnp.int32)
    for b in range(NV):
        cn = (cum[b] + off[b]) * inv_v
        offinv = off[b] * inv_v
        c0 = jnp.where(lane == 0, offinv, _perm(cn, lanem1))
        b1 = _perm(bn[b], lanep1)
        if b < NV - 1:
            b1 = jnp.where(lane == 15, _perm(bn[b + 1], lane0), b1)
        dC = cn - c0
        dB = b1 - bn[b]
        S = jnp.where(dC < 1e-5, dB, dB / dC)
        A = bn[b] - c0 * S
        sv[pl.ds(rrow + 16 * b, 16)] = S
        av[pl.ds(rrow + 16 * b, 16)] = A
        kb = (cn * 128.0 + 0.49999997).astype(jnp.int32)
        plsc.addupdate_scatter(hrow, [kb], ones)


def _ray_sample(r, ov, sv, av, hv):
    lane = lax.iota(jnp.int32, 16)
    lane_f = lane.astype(jnp.float32)
    lane15 = jnp.full((16,), 15, jnp.int32)

    rrow = r * NBIN
    hbase = r * HSTRIDE

    hcum = []
    for b in range(NV):
        hb = hv[pl.ds(hbase + 16 * b, 16)]
        hcum.append(plsc.cumsum(hb))
    hs = [_perm(hcum[b], lane15) for b in range(NV)]
    hoff = [jnp.zeros((16,), jnp.int32)]
    for b in range(NV - 1):
        hoff.append(hoff[b] + hs[b])

    srow_ref = sv.at[pl.ds(rrow, NBIN)]
    arow_ref = av.at[pl.ds(rrow, NBIN)]
    for b in range(NV):
        below = hcum[b] + hoff[b]
        Sg = plsc.load_gather(srow_ref, [below])
        Ag = plsc.load_gather(arow_ref, [below])
        ub = (lane_f + (16.0 * b + 0.5)) * (1.0 / 128.0)
        ov[pl.ds(r * NSAMP + 16 * b, 16)] = Ag + ub * Sg


@functools.partial(
    pl.kernel,
    out_type=jax.ShapeDtypeStruct((N_RAYS * NSAMP,), jnp.float32),
    mesh=plsc.VectorSubcoreMesh(core_axis_name="c", subcore_axis_name="s"),
    compiler_params=pltpu.CompilerParams(needs_layout_passes=False),
    scratch_types=[
        pltpu.VMEM((CHUNK * NWEIGHT + 16,), jnp.float32),
        pltpu.VMEM((CHUNK * NWEIGHT + 16,), jnp.float32),
        pltpu.VMEM((CHUNK * NBIN,), jnp.float32),
        pltpu.VMEM((CHUNK * NBIN,), jnp.float32),
        pltpu.VMEM((CHUNK * NSAMP,), jnp.float32),
        pltpu.VMEM((CHUNK * NSAMP,), jnp.float32),
        pltpu.VMEM((CHUNK * NBIN,), jnp.float32),
        pltpu.VMEM((CHUNK * NBIN,), jnp.float32),
        pltpu.VMEM((CHUNK * HSTRIDE,), jnp.int32),
        pltpu.SemaphoreType.DMA,
        pltpu.SemaphoreType.DMA,
        pltpu.SemaphoreType.DMA,
        pltpu.SemaphoreType.DMA,
    ],
)
def _sample_pdf_sc(b_hbm, w_hbm, o_hbm, wv0, wv1, bv0, bv1, ov0, ov1,
                   sv, av, hv, si0, si1, so0, so1):
    wid = lax.axis_index("s") * NUM_CORES + lax.axis_index("c")
    tile_base = wid * RAYS_PER_W
    bufs = ((wv0, bv0, ov0, si0, so0), (wv1, bv1, ov1, si1, so1))

    def in_slices(c):
        rbase = tile_base + c * CHUNK
        return (w_hbm.at[pl.ds(rbase * NWEIGHT, CHUNK * NWEIGHT)],
                b_hbm.at[pl.ds(rbase * NBIN, CHUNK * NBIN)])

    def out_slice(c):
        rbase = tile_base + c * CHUNK
        return o_hbm.at[pl.ds(rbase * NSAMP, CHUNK * NSAMP)]

    def start_in(c, wvx, bvx, sin):
        wh, bh = in_slices(c)
        pltpu.async_copy(wh, wvx.at[pl.ds(0, CHUNK * NWEIGHT)], sin)
        pltpu.async_copy(bh, bvx, sin)

    def wait_in(c, wvx, bvx, sin):
        wh, bh = in_slices(c)
        pltpu.make_async_copy(wh, wvx.at[pl.ds(0, CHUNK * NWEIGHT)], sin).wait()
        pltpu.make_async_copy(bh, bvx, sin).wait()

    start_in(0, wv0, bv0, si0)
    start_in(1, wv1, bv1, si1)

    def pair_body(g, carry):
        for par, (wvx, bvx, ovx, sin, sout) in enumerate(bufs):
            c = 2 * g + par
            wait_in(c, wvx, bvx, sin)

            @pl.when(c >= 2)
            def _drain_prev_out():
                pltpu.make_async_copy(ovx, out_slice(c - 2), sout).wait()

            @plsc.parallel_loop(0, CHUNK, unroll=4)
            def _rays_a(r):
                _ray_prep(r, wvx, bvx, sv, av, hv)

            @plsc.parallel_loop(0, CHUNK, unroll=8)
            def _rays_b(r):
                _ray_sample(r, ovx, sv, av, hv)

            pltpu.async_copy(ovx, out_slice(c), sout)

            @pl.when(c + 2 < NCHUNK)
            def _prefetch_next():
                start_in(c + 2, wvx, bvx, sin)
        return carry

    lax.fori_loop(0, NCHUNK // 2, pair_body, 0)

    pltpu.make_async_copy(ov0, out_slice(NCHUNK - 2), so0).wait()
    pltpu.make_async_copy(ov1, out_slice(NCHUNK - 1), so1).wait()


def kernel(bins, weights, n_samples):
    del n_samples
    out = _sample_pdf_sc(bins.reshape(-1), weights.reshape(-1))
    return out.reshape(N_RAYS, NSAMP)


# speedup vs baseline: 1.0772x; 1.0024x over previous
import functools

import jax
import jax.numpy as jnp
from jax import lax
from jax.experimental import pallas as pl
from jax.experimental.pallas import tpu as pltpu
from jax.experimental.pallas import tpu_sc as plsc

N_RAYS = 131072
NWEIGHT = 127
NBIN = 128
NSAMP = 128
NV = NBIN // 16

NUM_CORES = 2
NUM_SUBCORES = 16
NUM_W = NUM_CORES * NUM_SUBCORES
RAYS_PER_W = N_RAYS // NUM_W
CHUNK = 64
NCHUNK = RAYS_PER_W // CHUNK

HSTRIDE = 144

_GDN = lax.GatherDimensionNumbers(
    offset_dims=(), collapsed_slice_dims=(0,), start_index_map=(0,))


def _perm(x, idx):
    return lax.gather(x, idx[:, None], _GDN, (1,),
                      mode=lax.GatherScatterMode.PROMISE_IN_BOUNDS)


def _ray_prep(r, wv, bv, sv, av, hv):
    lane = lax.iota(jnp.int32, 16)
    zero_v = jnp.zeros((16,), jnp.float32)
    lane15 = jnp.full((16,), 15, jnp.int32)
    lane0 = jnp.zeros((16,), jnp.int32)
    lanem1 = jnp.maximum(lane - 1, 0)
    lanep1 = jnp.minimum(lane + 1, 15)

    wbase = r * NWEIGHT
    rrow = r * NBIN
    hbase = r * HSTRIDE

    w = [wv[pl.ds(wbase + 16 * b, 16)] + 1e-5 for b in range(NV)]
    w[NV - 1] = jnp.where(lane < 15, w[NV - 1], 0.0)

    cum = [plsc.cumsum(w[b]) for b in range(NV)]
    s = [_perm(cum[b], lane15) for b in range(NV)]
    off = [zero_v]
    for b in range(NV - 1):
        off.append(off[b] + s[b])
    tot_v = off[NV - 1] + s[NV - 1]
    inv_v = 1.0 / tot_v

    bn = [bv[pl.ds(rrow + 16 * b, 16)] for b in range(NV)]

    zi = jnp.zeros((16,), jnp.int32)
    for q in range(9):
        hv[pl.ds(hbase + 16 * q, 16)] = zi

    hrow = hv.at[pl.ds(hbase, HSTRIDE)]
    ones = jnp.ones((16,), jnp.int32)
    for b in range(NV):
        cn = (cum[b] + off[b]) * inv_v
        offinv = off[b] * inv_v
        c0 = jnp.where(lane == 0, offinv, _perm(cn, lanem1))
        b1 = _perm(bn[b], lanep1)
        if b < NV - 1:
            b1 = jnp.where(lane == 15, _perm(bn[b + 1], lane0), b1)
        dC = cn - c0
        dB = b1 - bn[b]
        S = jnp.where(dC < 1e-5, dB, dB / dC)
        A = bn[b] - c0 * S
        sv[pl.ds(rrow + 16 * b, 16)] = S
        av[pl.ds(rrow + 16 * b, 16)] = A
        kb = (cn * 128.0 + 0.49999997).astype(jnp.int32)
        plsc.addupdate_scatter(hrow, [kb], ones)


def _ray_sample(r, ov, sv, av, hv):
    lane = lax.iota(jnp.int32, 16)
    lane_f = lane.astype(jnp.float32)
    lane15 = jnp.full((16,), 15, jnp.int32)

    rrow = r * NBIN
    hbase = r * HSTRIDE

    hcum = []
    for b in range(NV):
        hb = hv[pl.ds(hbase + 16 * b, 16)]
        hcum.append(plsc.cumsum(hb))
    hs = [_perm(hcum[b], lane15) for b in range(NV)]
    hoff = [jnp.zeros((16,), jnp.int32)]
    for b in range(NV - 1):
        hoff.append(hoff[b] + hs[b])

    srow_ref = sv.at[pl.ds(rrow, NBIN)]
    arow_ref = av.at[pl.ds(rrow, NBIN)]
    for b in range(NV):
        below = hcum[b] + hoff[b]
        Sg = plsc.load_gather(srow_ref, [below])
        Ag = plsc.load_gather(arow_ref, [below])
        ub = (lane_f + (16.0 * b + 0.5)) * (1.0 / 128.0)
        ov[pl.ds(r * NSAMP + 16 * b, 16)] = Ag + ub * Sg


@functools.partial(
    pl.kernel,
    out_type=jax.ShapeDtypeStruct((N_RAYS * NSAMP,), jnp.float32),
    mesh=plsc.VectorSubcoreMesh(core_axis_name="c", subcore_axis_name="s"),
    compiler_params=pltpu.CompilerParams(needs_layout_passes=False),
    scratch_types=[
        pltpu.VMEM((CHUNK * NWEIGHT + 16,), jnp.float32),
        pltpu.VMEM((CHUNK * NWEIGHT + 16,), jnp.float32),
        pltpu.VMEM((CHUNK * NBIN,), jnp.float32),
        pltpu.VMEM((CHUNK * NBIN,), jnp.float32),
        pltpu.VMEM((CHUNK * NSAMP,), jnp.float32),
        pltpu.VMEM((CHUNK * NSAMP,), jnp.float32),
        pltpu.VMEM((CHUNK * NBIN,), jnp.float32),
        pltpu.VMEM((CHUNK * NBIN,), jnp.float32),
        pltpu.VMEM((CHUNK * HSTRIDE,), jnp.int32),
        pltpu.SemaphoreType.DMA,
        pltpu.SemaphoreType.DMA,
        pltpu.SemaphoreType.DMA,
        pltpu.SemaphoreType.DMA,
    ],
)
def _sample_pdf_sc(b_hbm, w_hbm, o_hbm, wv0, wv1, bv0, bv1, ov0, ov1,
                   sv, av, hv, si0, si1, so0, so1):
    wid = lax.axis_index("s") * NUM_CORES + lax.axis_index("c")
    tile_base = wid * RAYS_PER_W
    bufs = ((wv0, bv0, ov0, si0, so0), (wv1, bv1, ov1, si1, so1))

    def in_slices(c):
        rbase = tile_base + c * CHUNK
        return (w_hbm.at[pl.ds(rbase * NWEIGHT, CHUNK * NWEIGHT)],
                b_hbm.at[pl.ds(rbase * NBIN, CHUNK * NBIN)])

    def out_slice(c):
        rbase = tile_base + c * CHUNK
        return o_hbm.at[pl.ds(rbase * NSAMP, CHUNK * NSAMP)]

    def start_in(c, wvx, bvx, sin):
        wh, bh = in_slices(c)
        pltpu.async_copy(wh, wvx.at[pl.ds(0, CHUNK * NWEIGHT)], sin)
        pltpu.async_copy(bh, bvx, sin)

    def wait_in(c, wvx, bvx, sin):
        wh, bh = in_slices(c)
        pltpu.make_async_copy(wh, wvx.at[pl.ds(0, CHUNK * NWEIGHT)], sin).wait()
        pltpu.make_async_copy(bh, bvx, sin).wait()

    start_in(0, wv0, bv0, si0)
    start_in(1, wv1, bv1, si1)

    def pair_body(g, carry):
        for par, (wvx, bvx, ovx, sin, sout) in enumerate(bufs):
            c = 2 * g + par
            wait_in(c, wvx, bvx, sin)

            @pl.when(c >= 2)
            def _drain_prev_out():
                pltpu.make_async_copy(ovx, out_slice(c - 2), sout).wait()

            @plsc.parallel_loop(0, CHUNK, unroll=5)
            def _rays_a(r):
                _ray_prep(r, wvx, bvx, sv, av, hv)

            @plsc.parallel_loop(0, CHUNK, unroll=8)
            def _rays_b(r):
                _ray_sample(r, ovx, sv, av, hv)

            pltpu.async_copy(ovx, out_slice(c), sout)

            @pl.when(c + 2 < NCHUNK)
            def _prefetch_next():
                start_in(c + 2, wvx, bvx, sin)
        return carry

    lax.fori_loop(0, NCHUNK // 2, pair_body, 0)

    pltpu.make_async_copy(ov0, out_slice(NCHUNK - 2), so0).wait()
    pltpu.make_async_copy(ov1, out_slice(NCHUNK - 1), so1).wait()


def kernel(bins, weights, n_samples):
    del n_samples
    out = _sample_pdf_sc(bins.reshape(-1), weights.reshape(-1))
    return out.reshape(N_RAYS, NSAMP)
